# baseline (speedup 1.0000x reference)
import functools

import jax
import jax.numpy as jnp
from jax import lax
from jax.experimental import pallas as pl
from jax.experimental.pallas import tpu as pltpu
from jax.experimental.pallas import tpu_sc as plsc

B = 16384
D = 128

_info = plsc.get_sparse_core_info()
_NC, _NS = _info.num_cores, _info.num_subcores
_NW = _NC * _NS
_BPW = B // _NW
_CH = 128
_NCH = _BPW // _CH

_mesh = plsc.VectorSubcoreMesh(core_axis_name="c", subcore_axis_name="s")


@functools.partial(
    pl.kernel,
    out_type=jax.ShapeDtypeStruct((B, 2, D), jnp.float32),
    mesh=_mesh,
    scratch_types=[
        pltpu.VMEM((_NCH, _CH), jnp.int32),
        pltpu.VMEM((_BPW, D), jnp.float32),
        pltpu.SemaphoreType.DMA,
        pltpu.SemaphoreType.DMA,
    ],
)
def _first_layer_sc(loc_hbm, x_hbm, table_hbm, out_hbm, idx_v, emb_v, gsem, xsem):
    wid = lax.axis_index("s") * _NC + lax.axis_index("c")
    base = wid * _BPW

    xcopy = pltpu.async_copy(
        x_hbm.at[pl.ds(base, _BPW)], out_hbm.at[pl.ds(base, _BPW), 1], xsem
    )

    for j in range(_NCH):
        pltpu.sync_copy(loc_hbm.at[pl.ds(base + j * _CH, _CH)], idx_v.at[j])

    gathers = [
        pltpu.async_copy(
            table_hbm.at[idx_v.at[j]], emb_v.at[pl.ds(j * _CH, _CH)], gsem
        )
        for j in range(_NCH)
    ]

    for g in gathers:
        g.wait()
    pltpu.sync_copy(emb_v, out_hbm.at[pl.ds(base, _BPW), 0])
    xcopy.wait()


def kernel(loc, x, embedding_table):
    out3 = _first_layer_sc(loc.astype(jnp.int32), x, embedding_table)
    return out3.reshape(B, 2 * D)


# speedup vs baseline: 3.3678x; 3.3678x over previous
import functools

import jax
import jax.numpy as jnp
from jax import lax
from jax.experimental import pallas as pl
from jax.experimental.pallas import tpu as pltpu
from jax.experimental.pallas import tpu_sc as plsc

B = 16384
D = 128
VOCAB = 26

_info = plsc.get_sparse_core_info()
_NC, _NS = _info.num_cores, _info.num_subcores
_NW = _NC * _NS
_BPW = B // _NW
_CH = 128
_NCH = _BPW // _CH

_mesh = plsc.VectorSubcoreMesh(core_axis_name="c", subcore_axis_name="s")


@functools.partial(
    pl.kernel,
    out_type=jax.ShapeDtypeStruct((B, 2, D), jnp.float32),
    mesh=_mesh,
    scratch_types=[
        pltpu.VMEM_SHARED((VOCAB, D), jnp.float32),
        pltpu.VMEM((_NCH, _CH), jnp.int32),
        pltpu.VMEM((3, _CH, D), jnp.float32),
        pltpu.VMEM((_NCH, _CH, D), jnp.float32),
        pltpu.SemaphoreType.DMA,
        pltpu.SemaphoreType.DMA,
        pltpu.SemaphoreType.DMA,
        pltpu.SemaphoreType.DMA,
        pltpu.SemaphoreType.DMA,
        pltpu.SemaphoreType.DMA,
    ],
)
def _first_layer_sc(loc_hbm, x_hbm, table_hbm, out_hbm,
                    table_v, idx_v, emb_v, x_v,
                    tsem, isem, gsem, esem, xrsem, xwsem):
    sid = lax.axis_index("s")
    wid = sid * _NC + lax.axis_index("c")
    base = wid * _BPW

    @pl.when(sid == 0)
    def _stage_table():
        pltpu.async_copy(table_hbm, table_v, tsem).wait()

    idx_copies = [
        pltpu.async_copy(loc_hbm.at[pl.ds(base + j * _CH, _CH)],
                         idx_v.at[j], isem)
        for j in range(_NCH)
    ]
    xreads = [
        pltpu.async_copy(x_hbm.at[pl.ds(base + j * _CH, _CH)],
                         x_v.at[j], xrsem)
        for j in range(_NCH)
    ]

    for c in idx_copies:
        c.wait()
    plsc.subcore_barrier()
    gathers = {}
    for j in range(3):
        gathers[j] = pltpu.async_copy(
            table_hbm.at[idx_v.at[j]], emb_v.at[j], gsem)

    xwrites = []
    for j in range(_NCH):
        xreads[j].wait()
        xwrites.append(pltpu.async_copy(
            x_v.at[j], out_hbm.at[pl.ds(base + j * _CH, _CH), 1], xwsem))

    ewrites = {}
    for j in range(3):
        gathers[j].wait()
        ewrites[j] = pltpu.async_copy(
            emb_v.at[j], out_hbm.at[pl.ds(base + j * _CH, _CH), 0], esem)
    ewrites[0].wait()
    gathers[3] = pltpu.async_copy(
        table_hbm.at[idx_v.at[3]], emb_v.at[0], gsem)
    gathers[3].wait()
    ewrites[3] = pltpu.async_copy(
        emb_v.at[0], out_hbm.at[pl.ds(base + 3 * _CH, _CH), 0], esem)

    for w in xwrites:
        w.wait()
    for j in (1, 2, 3):
        ewrites[j].wait()


def kernel(loc, x, embedding_table):
    out3 = _first_layer_sc(loc.astype(jnp.int32), x, embedding_table)
    return out3.reshape(B, 2 * D)


# speedup vs baseline: 4.3384x; 1.2882x over previous
import functools

import jax
import jax.numpy as jnp
from jax import lax
from jax.experimental import pallas as pl
from jax.experimental.pallas import tpu as pltpu
from jax.experimental.pallas import tpu_sc as plsc

B = 16384
D = 128
VOCAB = 26

_info = plsc.get_sparse_core_info()
_NC, _NS = _info.num_cores, _info.num_subcores
_NW = _NC * _NS
_BPW = B // _NW
_CH = 128
_NCH = _BPW // _CH

_mesh = plsc.VectorSubcoreMesh(core_axis_name="c", subcore_axis_name="s")


@functools.partial(
    pl.kernel,
    out_type=jax.ShapeDtypeStruct((B, 2 * D), jnp.float32),
    mesh=_mesh,
    scratch_types=[
        pltpu.VMEM_SHARED((VOCAB, D), jnp.float32),
        pltpu.VMEM((_NCH, _CH), jnp.int32),
        pltpu.VMEM((3, _CH, D), jnp.float32),
        pltpu.VMEM((_NCH, _CH, D), jnp.float32),
        pltpu.SemaphoreType.DMA,
        pltpu.SemaphoreType.DMA,
        pltpu.SemaphoreType.DMA,
        pltpu.SemaphoreType.DMA,
        pltpu.SemaphoreType.DMA,
        pltpu.SemaphoreType.DMA,
    ],
)
def _first_layer_sc(loc_hbm, x_hbm, table_hbm, out_hbm,
                    table_v, idx_v, emb_v, x_v,
                    tsem, isem, gsem, esem, xrsem, xwsem):
    sid = lax.axis_index("s")
    wid = sid * _NC + lax.axis_index("c")
    base = wid * _BPW

    @pl.when(sid == 0)
    def _stage_table():
        pltpu.async_copy(table_hbm, table_v, tsem).wait()

    idx_copies = [
        pltpu.async_copy(loc_hbm.at[pl.ds(base + j * _CH, _CH)],
                         idx_v.at[j], isem)
        for j in range(_NCH)
    ]
    xreads = [
        pltpu.async_copy(x_hbm.at[pl.ds(base + j * _CH, _CH)],
                         x_v.at[j], xrsem)
        for j in range(_NCH)
    ]

    for c in idx_copies:
        c.wait()
    plsc.subcore_barrier()
    gathers = {}
    for j in range(3):
        gathers[j] = pltpu.async_copy(
            table_hbm.at[idx_v.at[j]], emb_v.at[j], gsem)

    xwrites = []
    for j in range(_NCH):
        xreads[j].wait()
        xwrites.append(pltpu.async_copy(
            x_v.at[j], out_hbm.at[pl.ds(base + j * _CH, _CH), pl.ds(D, D)], xwsem))

    ewrites = {}
    for j in range(3):
        gathers[j].wait()
        ewrites[j] = pltpu.async_copy(
            emb_v.at[j], out_hbm.at[pl.ds(base + j * _CH, _CH), pl.ds(0, D)], esem)
    ewrites[0].wait()
    gathers[3] = pltpu.async_copy(
        table_hbm.at[idx_v.at[3]], emb_v.at[0], gsem)
    gathers[3].wait()
    ewrites[3] = pltpu.async_copy(
        emb_v.at[0], out_hbm.at[pl.ds(base + 3 * _CH, _CH), pl.ds(0, D)], esem)

    for w in xwrites:
        w.wait()
    for j in (1, 2, 3):
        ewrites[j].wait()


def kernel(loc, x, embedding_table):
    return _first_layer_sc(loc.astype(jnp.int32), x, embedding_table)
